# baseline (speedup 1.0000x reference)
import functools

import jax
import jax.numpy as jnp
from jax import lax
from jax.experimental import pallas as pl
from jax.experimental.pallas import tpu as pltpu
from jax.experimental.pallas import tpu_sc as plsc

_NW = 32
_CHN = 512
_NBUF = 2


def _sc_gather_hmajor(table, indices_t):
    h, n = indices_t.shape
    d = table.shape[1]
    nb_n = n // _CHN
    n_chunks_total = nb_n * h
    chunks_per_w = n_chunks_total // _NW
    assert chunks_per_w % _NBUF == 0 and n_chunks_total % _NW == 0
    mesh = plsc.VectorSubcoreMesh(core_axis_name="c", subcore_axis_name="s")

    @functools.partial(
        pl.kernel,
        mesh=mesh,
        out_type=jax.ShapeDtypeStruct((h * n, d), jnp.float32),
        compiler_params=pltpu.CompilerParams(use_tc_tiling_on_sc=False),
        scratch_types=[
            pltpu.VMEM((_CHN,), jnp.int32),
            pltpu.VMEM((_CHN,), jnp.int32),
            pltpu.VMEM((_CHN, d), jnp.float32),
            pltpu.VMEM((_CHN, d), jnp.float32),
            pltpu.SemaphoreType.DMA,
            pltpu.SemaphoreType.DMA,
            pltpu.SemaphoreType.DMA,
            pltpu.SemaphoreType.DMA,
            pltpu.SemaphoreType.DMA,
            pltpu.SemaphoreType.DMA,
        ],
    )
    def k(table_hbm, idx_hbm, out_hbm,
          i0, i1, r0, r1, gi0, gi1, gg0, gg1, gs0, gs1):
        wid = lax.axis_index("s") * 2 + lax.axis_index("c")
        idx_v = (i0, i1)
        rows = (r0, r1)
        isem = (gi0, gi1)
        gsem = (gg0, gg1)
        ssem = (gs0, gs1)

        def coords(c):
            q = wid * chunks_per_w + c
            return q // nb_n, (q % nb_n) * _CHN

        def idx_desc(c, b):
            hb, nlo = coords(c)
            src = idx_hbm.at[hb, pl.ds(nlo, _CHN)]
            return pltpu.make_async_copy(src, idx_v[b], isem[b])

        def gather_desc(c, b):
            return pltpu.make_async_copy(
                table_hbm.at[idx_v[b]], rows[b], gsem[b])

        def store_desc(c, b):
            hb, nlo = coords(c)
            dst = out_hbm.at[pl.ds(hb * n + nlo, _CHN), :]
            return pltpu.make_async_copy(rows[b], dst, ssem[b])

        for b in range(_NBUF):
            idx_desc(b, b).start()
        for b in range(_NBUF):
            idx_desc(b, b).wait()
            gather_desc(b, b).start()
        for b in range(_NBUF):
            gather_desc(b, b).wait()
            idx_desc(b + _NBUF, b).start()
            store_desc(b, b).start()

        @pl.loop(_NBUF, chunks_per_w, step=_NBUF)
        def body(g):
            for b in range(_NBUF):
                c = g + b
                idx_desc(c, b).wait()
                store_desc(c - _NBUF, b).wait()
                gather_desc(c, b).start()
                gather_desc(c, b).wait()

                @pl.when(c + _NBUF < chunks_per_w)
                def _():
                    idx_desc(c + _NBUF, b).start()

                store_desc(c, b).start()

        for b in range(_NBUF):
            store_desc(chunks_per_w - _NBUF + b, b).wait()

    return k(table, indices_t)


def _tc_transpose(x, n, h, d):
    def body(x_ref, o_ref):
        o_ref[0] = jnp.transpose(x_ref[...], (1, 0))

    return pl.pallas_call(
        body,
        grid=(h,),
        in_specs=[pl.BlockSpec((n, d), lambda i: (i, 0))],
        out_specs=pl.BlockSpec((1, d, n), lambda i: (i, 0, 0)),
        out_shape=jax.ShapeDtypeStruct((h, d, n), jnp.float32),
    )(x)


def kernel(indices, table):
    n, h = indices.shape
    d = table.shape[1]
    x = _sc_gather_hmajor(table, indices.T)
    out3 = _tc_transpose(x, n, h, d)
    return out3.transpose(2, 0, 1)
